# baseline (speedup 1.0000x reference)
import jax
import jax.numpy as jnp
from jax import lax
from jax.experimental import pallas as pl
from jax.experimental.pallas import tpu as pltpu
from jax.experimental.pallas import tpu_sc as plsc

B = 16384
ED = 64
RD = 32
NC = 2
NS = 16
L = 16
NW = NC * NS
BPW = B // NW
GROUPS = BPW // L
IDXROWS = BPW // 128


def _issue(g, refs, bufs, midx, lane, sem):
    head_r, rel_r, tail_r, ent_r, remb_r, tmat_r = refs
    hidx, ridx, tidx, h_v, t_v, r_v, m_v = bufs
    p = g * L + lane
    prow = lax.shift_right_logical(p, 7)
    pcol = lax.bitwise_and(p, 127)
    hi = plsc.load_gather(hidx, [prow, pcol])
    ti = plsc.load_gather(tidx, [prow, pcol])
    ri = plsc.load_gather(ridx, [prow, pcol])
    copies = [
        pltpu.async_copy(ent_r.at[hi], h_v, sem),
        pltpu.async_copy(ent_r.at[ti], t_v, sem),
        pltpu.async_copy(remb_r.at[ri], r_v, sem),
        pltpu.async_copy(tmat_r.at[midx.at[g, 0]], m_v.at[pl.ds(0, 8 * L)], sem),
        pltpu.async_copy(tmat_r.at[midx.at[g, 1]], m_v.at[pl.ds(8 * L, 8 * L)], sem),
    ]
    return copies


def _wait(refs, bufs, sem):
    head_r, rel_r, tail_r, ent_r, remb_r, tmat_r = refs
    hidx, ridx, tidx, h_v, t_v, r_v, m_v = bufs
    pltpu.make_async_copy(ent_r.at[pl.ds(0, L)], h_v, sem).wait()
    pltpu.make_async_copy(ent_r.at[pl.ds(0, L)], t_v, sem).wait()
    pltpu.make_async_copy(remb_r.at[pl.ds(0, L)], r_v, sem).wait()
    pltpu.make_async_copy(
        tmat_r.at[pl.ds(0, 8 * L)], m_v.at[pl.ds(0, 8 * L)], sem).wait()
    pltpu.make_async_copy(
        tmat_r.at[pl.ds(0, 8 * L)], m_v.at[pl.ds(8 * L, 8 * L)], sem).wait()


def _compute(g, refs, bufs, lane, lane16, dT, score_v):
    hidx, ridx, tidx, h_v, t_v, r_v, m_v = bufs

    def dpre(i, carry):
        for u in range(8):
            dd = jnp.zeros((L,), jnp.int32) + (i * 8 + u)
            hT = plsc.load_gather(h_v, [lane, dd])
            tT = plsc.load_gather(t_v, [lane, dd])
            dT[i * 8 + u] = hT - tT
        return carry

    lax.fori_loop(0, ED // 8, dpre, 0)

    nrm = jnp.zeros((L,), jnp.float32)
    for jb in range(4):
        j0 = jb * 8
        rows = [lane16 + ((j0 + jj) >> 1) for jj in range(8)]

        def dstep(i, accs, rows=rows, j0=j0):
            out = list(accs)
            for u in range(4):
                d = i * 4 + u
                dvec = dT[d]
                c0 = jnp.zeros((L,), jnp.int32) + d
                c1 = c0 + ED
                for jj in range(8):
                    col = c1 if ((j0 + jj) & 1) else c0
                    m = plsc.load_gather(m_v, [rows[jj], col])
                    out[jj] = out[jj] + m * dvec
            return tuple(out)

        accs = lax.fori_loop(
            0, ED // 4, dstep,
            tuple(jnp.zeros((L,), jnp.float32) for _ in range(8)))
        for jj in range(8):
            rT = plsc.load_gather(r_v, [lane, jnp.full((L,), j0 + jj, jnp.int32)])
            sc = accs[jj] + rT
            nrm = nrm + sc * sc

    x = jnp.maximum(nrm, jnp.float32(1e-30))
    i = plsc.bitcast(x, jnp.int32)
    i = 0x5F3759DF - lax.shift_right_logical(i, 1)
    y = plsc.bitcast(i, jnp.float32)
    for _ in range(3):
        y = y * (jnp.float32(1.5) - jnp.float32(0.5) * x * y * y)
    res = -(x * y)
    srow = lax.shift_right_logical(g * L, 7)
    scol = lax.bitwise_and(g * L, 127)
    plsc.store_scatter(score_v, [jnp.full((L,), srow, jnp.int32),
                                 scol + lane], res)


def _body(head_r, rel_r, tail_r, ent_r, remb_r, tmat_r, out_r,
          hidx, ridx, tidx,
          h0, t0, r0, m0,
          h1, t1, r1, m1,
          midx, dT, score_v, sem0, sem1):
    c = lax.axis_index("c")
    s = lax.axis_index("s")
    wid = s * NC + c
    row0 = wid * IDXROWS

    pltpu.sync_copy(head_r.at[pl.ds(row0, IDXROWS)], hidx)
    pltpu.sync_copy(rel_r.at[pl.ds(row0, IDXROWS)], ridx)
    pltpu.sync_copy(tail_r.at[pl.ds(row0, IDXROWS)], tidx)

    lane = lax.iota(jnp.int32, L)
    lane16 = lane * L
    refs = (head_r, rel_r, tail_r, ent_r, remb_r, tmat_r)
    bufs0 = (hidx, ridx, tidx, h0, t0, r0, m0)
    bufs1 = (hidx, ridx, tidx, h1, t1, r1, m1)

    def buildm(g, carry):
        for s_ in range(L):
            ps = g * L + s_
            rs = plsc.load_gather(
                ridx,
                [jnp.full((L,), lax.shift_right_logical(ps, 7), jnp.int32),
                 jnp.full((L,), lax.bitwise_and(ps, 127), jnp.int32)])
            vals = (lax.shift_right_logical(rs, 3) * 128 + lane * 8
                    + lax.bitwise_and(rs, 7))
            midx[g, s_ // 8, pl.ds((s_ % 8) * L, L)] = vals
        return carry

    lax.fori_loop(0, GROUPS, buildm, 0)

    _issue(0, refs, bufs0, midx, lane, sem0)

    def step(gg, carry):
        g0 = gg * 2
        _issue(g0 + 1, refs, bufs1, midx, lane, sem1)
        _wait(refs, bufs0, sem0)
        _compute(g0, refs, bufs0, lane, lane16, dT, score_v)

        @pl.when(gg < GROUPS // 2 - 1)
        def _():
            _issue(g0 + 2, refs, bufs0, midx, lane, sem0)

        _wait(refs, bufs1, sem1)
        _compute(g0 + 1, refs, bufs1, lane, lane16, dT, score_v)
        return carry

    lax.fori_loop(0, GROUPS // 2, step, 0)
    pltpu.sync_copy(score_v, out_r.at[pl.ds(row0, IDXROWS)])


@jax.jit
def _transr_sc(head2, rel2, tail2, ent, remb, tmatp):
    mesh = plsc.VectorSubcoreMesh(
        core_axis_name="c", subcore_axis_name="s",
        num_cores=NC, num_subcores=NS)
    dbl = lambda: [
        pltpu.VMEM((L, ED), jnp.float32),
        pltpu.VMEM((L, ED), jnp.float32),
        pltpu.VMEM((L, RD), jnp.float32),
        pltpu.VMEM((16 * L, 128), jnp.float32),
    ]
    fn = pl.kernel(
        _body,
        out_type=jax.ShapeDtypeStruct((128, 128), jnp.float32),
        mesh=mesh,
        compiler_params=pltpu.CompilerParams(
            needs_layout_passes=False, use_tc_tiling_on_sc=False),
        scratch_types=[
            pltpu.VMEM((IDXROWS, 128), jnp.int32),
            pltpu.VMEM((IDXROWS, 128), jnp.int32),
            pltpu.VMEM((IDXROWS, 128), jnp.int32),
            *dbl(), *dbl(),
            pltpu.VMEM((GROUPS, 2, 128), jnp.int32),
            pltpu.VMEM((ED, L), jnp.float32),
            pltpu.VMEM((IDXROWS, 128), jnp.float32),
            pltpu.SemaphoreType.DMA,
            pltpu.SemaphoreType.DMA,
        ],
    )
    return fn(head2, rel2, tail2, ent, remb, tmatp)


def kernel(head, relation, tail, entity_emb, relation_emb, transfer_mat):
    tmatp = (transfer_mat.reshape(125, 8, 16, 128)
             .transpose(0, 2, 1, 3).reshape(16000, 128))
    out2 = _transr_sc(
        head.reshape(128, 128), relation.reshape(128, 128),
        tail.reshape(128, 128), entity_emb, relation_emb, tmatp)
    return out2.reshape(B)


# speedup vs baseline: 1.0580x; 1.0580x over previous
import jax
import jax.numpy as jnp
from jax import lax
from jax.experimental import pallas as pl
from jax.experimental.pallas import tpu as pltpu
from jax.experimental.pallas import tpu_sc as plsc

B = 16384
ED = 64
RD = 32
NC = 2
NS = 16
L = 16
NW = NC * NS
BPW = B // NW
GROUPS = BPW // L
IDXROWS = BPW // 128


def _issue(g, refs, bufs, midx, lane, sem):
    head_r, rel_r, tail_r, ent_r, remb_r, tmat_r = refs
    hidx, ridx, tidx, h_v, t_v, r_v, m_v = bufs
    p = g * L + lane
    prow = lax.shift_right_logical(p, 7)
    pcol = lax.bitwise_and(p, 127)
    hi = plsc.load_gather(hidx, [prow, pcol])
    ti = plsc.load_gather(tidx, [prow, pcol])
    ri = plsc.load_gather(ridx, [prow, pcol])
    copies = [
        pltpu.async_copy(ent_r.at[hi], h_v, sem),
        pltpu.async_copy(ent_r.at[ti], t_v, sem),
        pltpu.async_copy(remb_r.at[ri], r_v, sem),
        pltpu.async_copy(tmat_r.at[midx.at[g, 0]], m_v.at[pl.ds(0, 8 * L)], sem),
        pltpu.async_copy(tmat_r.at[midx.at[g, 1]], m_v.at[pl.ds(8 * L, 8 * L)], sem),
    ]
    return copies


def _wait(refs, bufs, sem):
    head_r, rel_r, tail_r, ent_r, remb_r, tmat_r = refs
    hidx, ridx, tidx, h_v, t_v, r_v, m_v = bufs
    pltpu.make_async_copy(ent_r.at[pl.ds(0, L)], h_v, sem).wait()
    pltpu.make_async_copy(ent_r.at[pl.ds(0, L)], t_v, sem).wait()
    pltpu.make_async_copy(remb_r.at[pl.ds(0, L)], r_v, sem).wait()
    pltpu.make_async_copy(
        tmat_r.at[pl.ds(0, 8 * L)], m_v.at[pl.ds(0, 8 * L)], sem).wait()
    pltpu.make_async_copy(
        tmat_r.at[pl.ds(0, 8 * L)], m_v.at[pl.ds(8 * L, 8 * L)], sem).wait()


def _compute(g, refs, bufs, lane, lane16, dT, score_v):
    hidx, ridx, tidx, h_v, t_v, r_v, m_v = bufs

    def dpre(i, carry):
        for u in range(8):
            dd = jnp.zeros((L,), jnp.int32) + (i * 8 + u)
            hT = plsc.load_gather(h_v, [lane, dd])
            tT = plsc.load_gather(t_v, [lane, dd])
            dT[i * 8 + u] = hT - tT
        return carry

    lax.fori_loop(0, ED // 8, dpre, 0)

    nrm = jnp.zeros((L,), jnp.float32)
    for jb in range(4):
        j0 = jb * 8
        rows = [lane16 + ((j0 + jj) >> 1) for jj in range(8)]

        def dstep(i, accs, rows=rows, j0=j0):
            out = list(accs)
            for u in range(4):
                d = i * 4 + u
                dvec = dT[d]
                c0 = jnp.zeros((L,), jnp.int32) + d
                c1 = c0 + ED
                for jj in range(8):
                    col = c1 if ((j0 + jj) & 1) else c0
                    m = plsc.load_gather(m_v, [rows[jj], col])
                    out[jj] = out[jj] + m * dvec
            return tuple(out)

        accs = lax.fori_loop(
            0, ED // 4, dstep,
            tuple(jnp.zeros((L,), jnp.float32) for _ in range(8)))
        for jj in range(8):
            rT = plsc.load_gather(r_v, [lane, jnp.full((L,), j0 + jj, jnp.int32)])
            sc = accs[jj] + rT
            nrm = nrm + sc * sc

    x = jnp.maximum(nrm, jnp.float32(1e-30))
    i = plsc.bitcast(x, jnp.int32)
    i = 0x5F3759DF - lax.shift_right_logical(i, 1)
    y = plsc.bitcast(i, jnp.float32)
    for _ in range(3):
        y = y * (jnp.float32(1.5) - jnp.float32(0.5) * x * y * y)
    res = -(x * y)
    srow = lax.shift_right_logical(g * L, 7)
    scol = lax.bitwise_and(g * L, 127)
    plsc.store_scatter(score_v, [jnp.full((L,), srow, jnp.int32),
                                 scol + lane], res)


def _body(head_r, rel_r, tail_r, ent_r, remb_r, tmat_r, out_r,
          hidx, ridx, tidx,
          h0, t0, r0, m0,
          h1, t1, r1, m1,
          midx, dT, score_v, sem0, sem1):
    c = lax.axis_index("c")
    s = lax.axis_index("s")
    wid = s * NC + c
    row0 = wid * IDXROWS

    pltpu.sync_copy(head_r.at[pl.ds(row0, IDXROWS)], hidx)
    pltpu.sync_copy(rel_r.at[pl.ds(row0, IDXROWS)], ridx)
    pltpu.sync_copy(tail_r.at[pl.ds(row0, IDXROWS)], tidx)

    lane = lax.iota(jnp.int32, L)
    lane16 = lane * L
    refs = (head_r, rel_r, tail_r, ent_r, remb_r, tmat_r)
    bufs0 = (hidx, ridx, tidx, h0, t0, r0, m0)
    bufs1 = (hidx, ridx, tidx, h1, t1, r1, m1)

    def buildm(g, carry):
        for s_ in range(L):
            ps = g * L + s_
            rs = plsc.load_gather(
                ridx,
                [jnp.full((L,), lax.shift_right_logical(ps, 7), jnp.int32),
                 jnp.full((L,), lax.bitwise_and(ps, 127), jnp.int32)])
            vals = (lax.shift_right_logical(rs, 3) * 128 + lane * 8
                    + lax.bitwise_and(rs, 7))
            midx[g, s_ // 8, pl.ds((s_ % 8) * L, L)] = vals
        return carry

    lax.fori_loop(0, GROUPS, buildm, 0)

    _issue(0, refs, bufs0, midx, lane, sem0)

    def step(gg, carry):
        g0 = gg * 2
        _issue(g0 + 1, refs, bufs1, midx, lane, sem1)
        _wait(refs, bufs0, sem0)
        _compute(g0, refs, bufs0, lane, lane16, dT, score_v)

        @pl.when(gg < GROUPS // 2 - 1)
        def _():
            _issue(g0 + 2, refs, bufs0, midx, lane, sem0)

        _wait(refs, bufs1, sem1)
        _compute(g0 + 1, refs, bufs1, lane, lane16, dT, score_v)
        return carry

    lax.fori_loop(0, GROUPS // 2, step, 0)
    pltpu.sync_copy(score_v, out_r.at[pl.ds(row0, IDXROWS)])


@jax.jit
def _transr_sc(head2, rel2, tail2, ent, remb, tmatp):
    mesh = plsc.VectorSubcoreMesh(
        core_axis_name="c", subcore_axis_name="s",
        num_cores=NC, num_subcores=NS)
    dbl = lambda: [
        pltpu.VMEM((L, 128), jnp.float32),
        pltpu.VMEM((L, 128), jnp.float32),
        pltpu.VMEM((L, RD), jnp.float32),
        pltpu.VMEM((16 * L, 128), jnp.float32),
    ]
    fn = pl.kernel(
        _body,
        out_type=jax.ShapeDtypeStruct((128, 128), jnp.float32),
        mesh=mesh,
        compiler_params=pltpu.CompilerParams(
            needs_layout_passes=False, use_tc_tiling_on_sc=False),
        scratch_types=[
            pltpu.VMEM((IDXROWS, 128), jnp.int32),
            pltpu.VMEM((IDXROWS, 128), jnp.int32),
            pltpu.VMEM((IDXROWS, 128), jnp.int32),
            *dbl(), *dbl(),
            pltpu.VMEM((GROUPS, 2, 128), jnp.int32),
            pltpu.VMEM((ED, L), jnp.float32),
            pltpu.VMEM((IDXROWS, 128), jnp.float32),
            pltpu.SemaphoreType.DMA,
            pltpu.SemaphoreType.DMA,
        ],
    )
    return fn(head2, rel2, tail2, ent, remb, tmatp)


def kernel(head, relation, tail, entity_emb, relation_emb, transfer_mat):
    tmatp = (transfer_mat.reshape(125, 8, 16, 128)
             .transpose(0, 2, 1, 3).reshape(16000, 128))
    entp = jnp.pad(entity_emb, ((0, 0), (0, 128 - ED)))
    out2 = _transr_sc(
        head.reshape(128, 128), relation.reshape(128, 128),
        tail.reshape(128, 128), entp, relation_emb, tmatp)
    return out2.reshape(B)


# speedup vs baseline: 1.0649x; 1.0065x over previous
import jax
import jax.numpy as jnp
from jax import lax
from jax.experimental import pallas as pl
from jax.experimental.pallas import tpu as pltpu
from jax.experimental.pallas import tpu_sc as plsc

B = 16384
ED = 64
RD = 32
NC = 2
NS = 16
L = 16
NW = NC * NS
BPW = B // NW
GROUPS = BPW // L
IDXROWS = BPW // 128


def _issue(g, refs, bufs, midx, lane, sem):
    head_r, rel_r, tail_r, ent_r, remb_r, tmat_r = refs
    hidx, ridx, tidx, h_v, t_v, r_v, m_v = bufs
    p = g * L + lane
    prow = lax.shift_right_logical(p, 7)
    pcol = lax.bitwise_and(p, 127)
    hi = plsc.load_gather(hidx, [prow, pcol])
    ti = plsc.load_gather(tidx, [prow, pcol])
    ri = plsc.load_gather(ridx, [prow, pcol])
    copies = [
        pltpu.async_copy(ent_r.at[hi], h_v, sem),
        pltpu.async_copy(ent_r.at[ti], t_v, sem),
        pltpu.async_copy(remb_r.at[ri], r_v, sem),
        pltpu.async_copy(tmat_r.at[midx.at[g, 0]], m_v.at[pl.ds(0, 8 * L)], sem),
        pltpu.async_copy(tmat_r.at[midx.at[g, 1]], m_v.at[pl.ds(8 * L, 8 * L)], sem),
    ]
    return copies


def _wait(refs, bufs, sem):
    head_r, rel_r, tail_r, ent_r, remb_r, tmat_r = refs
    hidx, ridx, tidx, h_v, t_v, r_v, m_v = bufs
    pltpu.make_async_copy(ent_r.at[pl.ds(0, L)], h_v, sem).wait()
    pltpu.make_async_copy(ent_r.at[pl.ds(0, L)], t_v, sem).wait()
    pltpu.make_async_copy(remb_r.at[pl.ds(0, L)], r_v, sem).wait()
    pltpu.make_async_copy(
        tmat_r.at[pl.ds(0, 8 * L)], m_v.at[pl.ds(0, 8 * L)], sem).wait()
    pltpu.make_async_copy(
        tmat_r.at[pl.ds(0, 8 * L)], m_v.at[pl.ds(8 * L, 8 * L)], sem).wait()


def _compute(g, refs, bufs, lane, lane16, dT, score_v):
    hidx, ridx, tidx, h_v, t_v, r_v, m_v = bufs

    def dpre(i, carry):
        for u in range(8):
            dd = jnp.zeros((L,), jnp.int32) + (i * 8 + u)
            hT = plsc.load_gather(h_v, [lane, dd])
            tT = plsc.load_gather(t_v, [lane, dd])
            dT[i * 8 + u] = hT - tT
        return carry

    lax.fori_loop(0, ED // 8, dpre, 0)

    nrm = jnp.zeros((L,), jnp.float32)
    for jb in range(4):
        j0 = jb * 8
        rows = [lane + ((j0 + jj) >> 1) * L for jj in range(8)]

        def dstep(i, accs, rows=rows, j0=j0):
            out = list(accs)
            for u in range(4):
                d = i * 4 + u
                dvec = dT[d]
                c0 = jnp.zeros((L,), jnp.int32) + d
                c1 = c0 + ED
                for jj in range(8):
                    col = c1 if ((j0 + jj) & 1) else c0
                    m = plsc.load_gather(m_v, [rows[jj], col])
                    out[jj] = out[jj] + m * dvec
            return tuple(out)

        accs = lax.fori_loop(
            0, ED // 4, dstep,
            tuple(jnp.zeros((L,), jnp.float32) for _ in range(8)))
        for jj in range(8):
            rT = plsc.load_gather(r_v, [lane, jnp.full((L,), j0 + jj, jnp.int32)])
            sc = accs[jj] + rT
            nrm = nrm + sc * sc

    x = jnp.maximum(nrm, jnp.float32(1e-30))
    i = plsc.bitcast(x, jnp.int32)
    i = 0x5F3759DF - lax.shift_right_logical(i, 1)
    y = plsc.bitcast(i, jnp.float32)
    for _ in range(3):
        y = y * (jnp.float32(1.5) - jnp.float32(0.5) * x * y * y)
    res = -(x * y)
    srow = lax.shift_right_logical(g * L, 7)
    scol = lax.bitwise_and(g * L, 127)
    plsc.store_scatter(score_v, [jnp.full((L,), srow, jnp.int32),
                                 scol + lane], res)


def _body(head_r, rel_r, tail_r, ent_r, remb_r, tmat_r, out_r,
          hidx, ridx, tidx,
          h0, t0, r0, m0,
          h1, t1, r1, m1,
          midx, dT, score_v, sem0, sem1):
    c = lax.axis_index("c")
    s = lax.axis_index("s")
    wid = s * NC + c
    row0 = wid * IDXROWS

    pltpu.sync_copy(head_r.at[pl.ds(row0, IDXROWS)], hidx)
    pltpu.sync_copy(rel_r.at[pl.ds(row0, IDXROWS)], ridx)
    pltpu.sync_copy(tail_r.at[pl.ds(row0, IDXROWS)], tidx)

    lane = lax.iota(jnp.int32, L)
    lane16 = lane * L
    refs = (head_r, rel_r, tail_r, ent_r, remb_r, tmat_r)
    bufs0 = (hidx, ridx, tidx, h0, t0, r0, m0)
    bufs1 = (hidx, ridx, tidx, h1, t1, r1, m1)

    def buildm(g, carry):
        p = g * L + lane
        rs = plsc.load_gather(
            ridx, [lax.shift_right_logical(p, 7), lax.bitwise_and(p, 127)])
        base = (lax.shift_right_logical(rs, 3) * 128 + lax.bitwise_and(rs, 7))
        for c_ in range(L):
            midx[g, c_ // 8, pl.ds((c_ % 8) * L, L)] = base + c_ * 8
        return carry

    lax.fori_loop(0, GROUPS, buildm, 0)

    _issue(0, refs, bufs0, midx, lane, sem0)

    def step(gg, carry):
        g0 = gg * 2
        _issue(g0 + 1, refs, bufs1, midx, lane, sem1)
        _wait(refs, bufs0, sem0)
        _compute(g0, refs, bufs0, lane, lane16, dT, score_v)

        @pl.when(gg < GROUPS // 2 - 1)
        def _():
            _issue(g0 + 2, refs, bufs0, midx, lane, sem0)

        _wait(refs, bufs1, sem1)
        _compute(g0 + 1, refs, bufs1, lane, lane16, dT, score_v)
        return carry

    lax.fori_loop(0, GROUPS // 2, step, 0)
    pltpu.sync_copy(score_v, out_r.at[pl.ds(row0, IDXROWS)])


@jax.jit
def _transr_sc(head2, rel2, tail2, ent, remb, tmatp):
    mesh = plsc.VectorSubcoreMesh(
        core_axis_name="c", subcore_axis_name="s",
        num_cores=NC, num_subcores=NS)
    dbl = lambda: [
        pltpu.VMEM((L, 128), jnp.float32),
        pltpu.VMEM((L, 128), jnp.float32),
        pltpu.VMEM((L, RD), jnp.float32),
        pltpu.VMEM((16 * L, 128), jnp.float32),
    ]
    fn = pl.kernel(
        _body,
        out_type=jax.ShapeDtypeStruct((128, 128), jnp.float32),
        mesh=mesh,
        compiler_params=pltpu.CompilerParams(
            needs_layout_passes=False, use_tc_tiling_on_sc=False),
        scratch_types=[
            pltpu.VMEM((IDXROWS, 128), jnp.int32),
            pltpu.VMEM((IDXROWS, 128), jnp.int32),
            pltpu.VMEM((IDXROWS, 128), jnp.int32),
            *dbl(), *dbl(),
            pltpu.VMEM((GROUPS, 2, 128), jnp.int32),
            pltpu.VMEM((ED, L), jnp.float32),
            pltpu.VMEM((IDXROWS, 128), jnp.float32),
            pltpu.SemaphoreType.DMA,
            pltpu.SemaphoreType.DMA,
        ],
    )
    return fn(head2, rel2, tail2, ent, remb, tmatp)


def kernel(head, relation, tail, entity_emb, relation_emb, transfer_mat):
    tmatp = (transfer_mat.reshape(125, 8, 16, 128)
             .transpose(0, 2, 1, 3).reshape(16000, 128))
    entp = jnp.pad(entity_emb, ((0, 0), (0, 128 - ED)))
    out2 = _transr_sc(
        head.reshape(128, 128), relation.reshape(128, 128),
        tail.reshape(128, 128), entp, relation_emb, tmatp)
    return out2.reshape(B)


# speedup vs baseline: 1.2653x; 1.1882x over previous
import jax
import jax.numpy as jnp
from jax import lax
from jax.experimental import pallas as pl
from jax.experimental.pallas import tpu as pltpu
from jax.experimental.pallas import tpu_sc as plsc

B = 16384
ED = 64
RD = 32
NC = 2
NS = 16
L = 16
NW = NC * NS
BPW = B // NW
GROUPS = BPW // L
IDXROWS = BPW // 128


def _issue(g, refs, bufs, midx, lane, sem):
    head_r, rel_r, tail_r, ent_r, remb_r, tmat_r = refs
    hidx, ridx, tidx, h_v, t_v, r_v, m_v = bufs
    p = g * L + lane
    prow = lax.shift_right_logical(p, 7)
    pcol = lax.bitwise_and(p, 127)
    hi = plsc.load_gather(hidx, [prow, pcol])
    ti = plsc.load_gather(tidx, [prow, pcol])
    ri = plsc.load_gather(ridx, [prow, pcol])
    copies = [
        pltpu.async_copy(ent_r.at[hi], h_v, sem),
        pltpu.async_copy(ent_r.at[ti], t_v, sem),
        pltpu.async_copy(remb_r.at[ri], r_v, sem),
        pltpu.async_copy(tmat_r.at[midx.at[g, 0]], m_v, sem),
    ]
    return copies


def _wait(refs, bufs, sem):
    head_r, rel_r, tail_r, ent_r, remb_r, tmat_r = refs
    hidx, ridx, tidx, h_v, t_v, r_v, m_v = bufs
    pltpu.make_async_copy(ent_r.at[pl.ds(0, L)], h_v, sem).wait()
    pltpu.make_async_copy(ent_r.at[pl.ds(0, L)], t_v, sem).wait()
    pltpu.make_async_copy(remb_r.at[pl.ds(0, L)], r_v, sem).wait()
    pltpu.make_async_copy(tmat_r.at[pl.ds(0, 8 * L)], m_v, sem).wait()


def _compute(g, refs, bufs, lane, lane16, dT, score_v):
    hidx, ridx, tidx, h_v, t_v, r_v, m_v = bufs

    def dpre(i, carry):
        for u in range(8):
            dd = jnp.zeros((L,), jnp.int32) + (i * 8 + u)
            hT = plsc.load_gather(h_v, [lane, dd])
            tT = plsc.load_gather(t_v, [lane, dd])
            dT[i * 8 + u] = hT - tT
        return carry

    lax.fori_loop(0, ED // 8, dpre, 0)

    nrm = jnp.zeros((L,), jnp.float32)
    for jb in range(4):
        j0 = jb * 8
        rows = [lane + (((j0 + jj) >> 2) * L) for jj in range(8)]

        def dstep(i, accs, rows=rows, j0=j0):
            out = list(accs)
            for u in range(4):
                d2 = i * 4 + u
                dv0 = dT[2 * d2]
                dv1 = dT[2 * d2 + 1]
                c0 = jnp.zeros((L,), jnp.int32) + d2
                for jj in range(8):
                    col = c0 + (((j0 + jj) & 3) * RD)
                    mi = plsc.load_gather(m_v, [rows[jj], col])
                    a, b = plsc.unpack(plsc.bitcast(mi, jnp.bfloat16),
                                       format=plsc.PackFormat.INTERLEAVED)
                    out[jj] = out[jj] + a * dv0 + b * dv1
            return tuple(out)

        accs = lax.fori_loop(
            0, ED // 8, dstep,
            tuple(jnp.zeros((L,), jnp.float32) for _ in range(8)))
        for jj in range(8):
            rT = plsc.load_gather(r_v, [lane, jnp.full((L,), j0 + jj, jnp.int32)])
            sc = accs[jj] + rT
            nrm = nrm + sc * sc

    x = jnp.maximum(nrm, jnp.float32(1e-30))
    i = plsc.bitcast(x, jnp.int32)
    i = 0x5F3759DF - lax.shift_right_logical(i, 1)
    y = plsc.bitcast(i, jnp.float32)
    for _ in range(3):
        y = y * (jnp.float32(1.5) - jnp.float32(0.5) * x * y * y)
    res = -(x * y)
    srow = lax.shift_right_logical(g * L, 7)
    scol = lax.bitwise_and(g * L, 127)
    plsc.store_scatter(score_v, [jnp.full((L,), srow, jnp.int32),
                                 scol + lane], res)


def _body(head_r, rel_r, tail_r, ent_r, remb_r, tmat_r, out_r,
          hidx, ridx, tidx,
          h0, t0, r0, m0,
          h1, t1, r1, m1,
          midx, dT, score_v, sem0, sem1):
    c = lax.axis_index("c")
    s = lax.axis_index("s")
    wid = s * NC + c
    row0 = wid * IDXROWS

    pltpu.sync_copy(head_r.at[pl.ds(row0, IDXROWS)], hidx)
    pltpu.sync_copy(rel_r.at[pl.ds(row0, IDXROWS)], ridx)
    pltpu.sync_copy(tail_r.at[pl.ds(row0, IDXROWS)], tidx)

    lane = lax.iota(jnp.int32, L)
    lane16 = lane * L
    refs = (head_r, rel_r, tail_r, ent_r, remb_r, tmat_r)
    bufs0 = (hidx, ridx, tidx, h0, t0, r0, m0)
    bufs1 = (hidx, ridx, tidx, h1, t1, r1, m1)

    def buildm(g, carry):
        p = g * L + lane
        rs = plsc.load_gather(
            ridx, [lax.shift_right_logical(p, 7), lax.bitwise_and(p, 127)])
        base = rs * 8
        for c_ in range(8):
            midx[g, c_ // 8, pl.ds((c_ % 8) * L, L)] = base + c_
        return carry

    lax.fori_loop(0, GROUPS, buildm, 0)

    _issue(0, refs, bufs0, midx, lane, sem0)

    def step(gg, carry):
        g0 = gg * 2
        _issue(g0 + 1, refs, bufs1, midx, lane, sem1)
        _wait(refs, bufs0, sem0)
        _compute(g0, refs, bufs0, lane, lane16, dT, score_v)

        @pl.when(gg < GROUPS // 2 - 1)
        def _():
            _issue(g0 + 2, refs, bufs0, midx, lane, sem0)

        _wait(refs, bufs1, sem1)
        _compute(g0 + 1, refs, bufs1, lane, lane16, dT, score_v)
        return carry

    lax.fori_loop(0, GROUPS // 2, step, 0)
    pltpu.sync_copy(score_v, out_r.at[pl.ds(row0, IDXROWS)])


@jax.jit
def _transr_sc(head2, rel2, tail2, ent, remb, tmatp):
    mesh = plsc.VectorSubcoreMesh(
        core_axis_name="c", subcore_axis_name="s",
        num_cores=NC, num_subcores=NS)
    dbl = lambda: [
        pltpu.VMEM((L, 128), jnp.float32),
        pltpu.VMEM((L, 128), jnp.float32),
        pltpu.VMEM((L, RD), jnp.float32),
        pltpu.VMEM((8 * L, 128), jnp.int32),
    ]
    fn = pl.kernel(
        _body,
        out_type=jax.ShapeDtypeStruct((128, 128), jnp.float32),
        mesh=mesh,
        compiler_params=pltpu.CompilerParams(
            needs_layout_passes=False, use_tc_tiling_on_sc=False),
        scratch_types=[
            pltpu.VMEM((IDXROWS, 128), jnp.int32),
            pltpu.VMEM((IDXROWS, 128), jnp.int32),
            pltpu.VMEM((IDXROWS, 128), jnp.int32),
            *dbl(), *dbl(),
            pltpu.VMEM((GROUPS, 1, 128), jnp.int32),
            pltpu.VMEM((ED, L), jnp.float32),
            pltpu.VMEM((IDXROWS, 128), jnp.float32),
            pltpu.SemaphoreType.DMA,
            pltpu.SemaphoreType.DMA,
        ],
    )
    return fn(head2, rel2, tail2, ent, remb, tmatp)


def kernel(head, relation, tail, entity_emb, relation_emb, transfer_mat):
    tm16 = transfer_mat.astype(jnp.bfloat16)
    tmi = jax.lax.bitcast_convert_type(
        tm16.reshape(1000, 1024, 2), jnp.int32)
    tmatp = tmi.reshape(8000, 128)
    entp = jnp.pad(entity_emb, ((0, 0), (0, 128 - ED)))
    out2 = _transr_sc(
        head.reshape(128, 128), relation.reshape(128, 128),
        tail.reshape(128, 128), entp, relation_emb, tmatp)
    return out2.reshape(B)
